# baseline (speedup 1.0000x reference)
import functools

import jax
import jax.numpy as jnp
from jax.experimental import pallas as pl
from jax.experimental.pallas import tpu as pltpu

MB = 16
BF = jnp.bfloat16


def _leaky(x):
    return jnp.maximum(x, 0.01 * x)


def _elu(x):
    return jnp.where(x > 0, x, jnp.exp(x) - 1.0)


def _bmm(a, b):
    return jax.lax.dot_general(
        a, b, (((2,), (1,)), ((0,), (0,))),
        preferred_element_type=jnp.float32)


def _mm(a, b):
    return jnp.dot(a, b, preferred_element_type=jnp.float32)


def _gru(x, h, wxr, wxz, wxn, whr, whz, whn, bxr, bxz, bxn, bhr, bhz, bhn):
    xb = x.astype(BF)
    hb = h.astype(BF)
    r = jax.nn.sigmoid(_mm(xb, wxr) + bxr + _mm(hb, whr) + bhr)
    z = jax.nn.sigmoid(_mm(xb, wxz) + bxz + _mm(hb, whz) + bhz)
    n = jnp.tanh(_mm(xb, wxn) + bxn + r * (_mm(hb, whn) + bhn))
    return (1.0 - z) * n + z * h


def _fused_body(
    atom_ref, bond_ref, aidxt_ref, bidxt_ref, mask_ref,
    wf_ref, bf_ref, wna_ref, wnb_ref, bn_ref,
    a0w1_ref, a0w2_ref, a0b_ref, a1w1_ref, a1w2_ref, a1b_ref,
    t0w_ref, t0b_ref, t1w_ref, t1b_ref,
    g0xr_ref, g0xz_ref, g0xn_ref, g0hr_ref, g0hz_ref, g0hn_ref,
    g0bxr_ref, g0bxz_ref, g0bxn_ref, g0bhr_ref, g0bhz_ref, g0bhn_ref,
    g1xr_ref, g1xz_ref, g1xn_ref, g1hr_ref, g1hz_ref, g1hn_ref,
    g1bxr_ref, g1bxz_ref, g1bxn_ref, g1bhr_ref, g1bhz_ref, g1bhn_ref,
    mw1_ref, mw2_ref, mb_ref, mtw_ref, mtb_ref,
    mgxr_ref, mgxz_ref, mgxn_ref, mghr_ref, mghz_ref, mghn_ref,
    mgbxr_ref, mgbxz_ref, mgbxn_ref, mgbhr_ref, mgbhz_ref, mgbhn_ref,
    ow_ref, ob_ref,
    af_out_ref, pred_out_ref, molfeat_out_ref,
    *, mb, L, NBR, NB,
):
    R = mb * L
    RN = mb * L * NBR
    atom = atom_ref[...].reshape(R, atom_ref.shape[-1]).astype(BF)
    bond = bond_ref[...].reshape(mb * NB, bond_ref.shape[-1]).astype(BF)
    aidxt = aidxt_ref[...]
    bidxt = bidxt_ref[...]
    mask = mask_ref[...]

    af = _leaky(_mm(atom, wf_ref[...]) + bf_ref[...])
    P = _mm(atom, wna_ref[...]).astype(BF)
    Q = _mm(bond, wnb_ref[...]).astype(BF)
    P3 = P.reshape(mb, L, 200)
    Q3 = Q.reshape(mb, NB, 200)

    iota_a = jax.lax.broadcasted_iota(jnp.int32, (1, L), 1)
    iota_b = jax.lax.broadcasted_iota(jnp.int32, (1, NB), 1)

    oha = (aidxt == iota_a).astype(BF)
    ohb = (bidxt == iota_b).astype(BF)
    oha3 = oha.reshape(mb, L * NBR, L)
    pad = (aidxt == (L - 1))
    smask3 = jnp.where(pad, -9e8, 0.0).reshape(mb, L * NBR, 1)
    amask3 = jnp.where(pad, 0.0, 1.0).reshape(mb, L * NBR, 1)

    nfP = _bmm(oha3, P3)
    nfQ = _bmm(ohb.reshape(mb, L * NBR, NB), Q3)
    nf_all = _leaky(nfP + nfQ + bn_ref[...])
    nf_b = nf_all.astype(BF)
    s2_all = _mm(nf_b.reshape(RN, 200), a0w2_ref[...]).reshape(mb, L * NBR, 1)

    s1 = _mm(af.astype(BF), a0w1_ref[...])
    b0 = a0b_ref[0, 0]

    def slc(x3, j, w):
        return x3[:, j * L:(j + 1) * L, :].reshape(R, w)

    sc = [_leaky(s1 + slc(s2_all, j, 1) + b0) + slc(smask3, j, 1)
          for j in range(NBR)]
    mx = functools.reduce(jnp.maximum, sc)
    e = [jnp.exp(sc[j] - mx) for j in range(NBR)]
    z = functools.reduce(jnp.add, e)
    attn = [e[j] / z * slc(amask3, j, 1) for j in range(NBR)]
    nf3 = nf_all.reshape(mb, L * NBR, 200)
    ws = functools.reduce(
        jnp.add, [attn[j] * slc(nf3, j, 200) for j in range(NBR)])
    wsum = functools.reduce(jnp.add, attn)
    ctx = _elu(_mm(ws.astype(BF), t0w_ref[...]) + wsum * t0b_ref[...])

    h1 = _gru(ctx, af,
              g0xr_ref[...], g0xz_ref[...], g0xn_ref[...],
              g0hr_ref[...], g0hz_ref[...], g0hn_ref[...],
              g0bxr_ref[...], g0bxz_ref[...], g0bxn_ref[...],
              g0bhr_ref[...], g0bhz_ref[...], g0bhn_ref[...])
    act = jnp.maximum(h1, 0.0)
    act_b = act.astype(BF)
    act3 = act_b.reshape(mb, L, 200)

    s1b = _mm(act_b, a1w1_ref[...])
    u = _mm(act_b, a1w2_ref[...]).astype(BF)
    su_all = _bmm(oha3, u.reshape(mb, L, 1))
    b1 = a1b_ref[0, 0]
    sc2 = [_leaky(s1b + slc(su_all, j, 1) + b1) + slc(smask3, j, 1)
           for j in range(NBR)]
    mx2 = functools.reduce(jnp.maximum, sc2)
    e2 = [jnp.exp(sc2[j] - mx2) for j in range(NBR)]
    z2 = functools.reduce(jnp.add, e2)
    attn2 = [e2[j] / z2 * slc(amask3, j, 1) for j in range(NBR)]
    oha_r = oha.reshape(mb, L * NBR, L)
    A2 = functools.reduce(
        jnp.add, [attn2[j] * slc(oha_r, j, L) for j in range(NBR)])
    ws2 = _bmm(A2.astype(BF).reshape(mb, L, L), act3).reshape(R, 200)
    wsum2 = functools.reduce(jnp.add, attn2)
    ctx2 = _elu(_mm(ws2.astype(BF), t1w_ref[...]) + wsum2 * t1b_ref[...])

    h2 = _gru(ctx2, h1,
              g1xr_ref[...], g1xz_ref[...], g1xn_ref[...],
              g1hr_ref[...], g1hz_ref[...], g1hn_ref[...],
              g1bxr_ref[...], g1bxz_ref[...], g1bxn_ref[...],
              g1bhr_ref[...], g1bhz_ref[...], g1bhn_ref[...])
    af_out_ref[...] = h2.reshape(mb, L, 200)

    act2 = jnp.maximum(h2, 0.0)
    act2_b = act2.astype(BF)
    act2_3 = act2.reshape(mb, L, 200)
    molfeat = jnp.sum((act2 * mask).reshape(mb, L, 200), axis=1)
    mmask_s = jnp.where(mask == 0, -9e8, 0.0)
    s2m = _mm(act2_b, mw2_ref[...])
    bm = mb_ref[0, 0]
    am = jnp.maximum(molfeat, 0.0)

    for _ in range(2):
        s1m = _mm(am.astype(BF), mw1_ref[...])
        s1m_b = jnp.broadcast_to(s1m.reshape(mb, 1, 1), (mb, L, 1)).reshape(R, 1)
        scm = _leaky(s1m_b + s2m + bm) + mmask_s
        scm3 = scm.reshape(mb, L, 1)
        mxm = jnp.max(scm3, axis=1, keepdims=True)
        em = jnp.exp(scm3 - mxm)
        zm = jnp.sum(em, axis=1, keepdims=True)
        attnm = em / zm * mask.reshape(mb, L, 1)
        wsm = jnp.sum(attnm * act2_3, axis=1)
        wsumm = jnp.sum(attnm, axis=1)
        ctxm = _elu(_mm(wsm.astype(BF), mtw_ref[...]) + wsumm * mtb_ref[...])
        molfeat = _gru(ctxm, molfeat,
                       mgxr_ref[...], mgxz_ref[...], mgxn_ref[...],
                       mghr_ref[...], mghz_ref[...], mghn_ref[...],
                       mgbxr_ref[...], mgbxz_ref[...], mgbxn_ref[...],
                       mgbhr_ref[...], mgbhz_ref[...], mgbhn_ref[...])
        am = jnp.maximum(molfeat, 0.0)

    pred_out_ref[...] = _mm(molfeat.astype(BF), ow_ref[...]) + ob_ref[...]
    molfeat_out_ref[...] = molfeat


def kernel(atom_list, bond_list, atom_degree_list, bond_degree_list, atom_mask, params):
    Bz, L, AD = atom_list.shape
    _, NB, BD = bond_list.shape
    NBR = atom_degree_list.shape[-1]
    FP = params["atom_fc"]["W"].shape[0]
    mb = MB
    grid = Bz // mb

    f32 = jnp.float32
    aidxt = atom_degree_list.astype(jnp.int32).transpose(0, 2, 1).reshape(-1, 1)
    bidxt = bond_degree_list.astype(jnp.int32).transpose(0, 2, 1).reshape(-1, 1)
    mask = atom_mask.astype(f32).reshape(Bz * L, 1)

    def lin_w(p):
        return p["W"].T.astype(BF)

    def row(b):
        return b.reshape(1, -1).astype(f32)

    def gru_parts(g):
        Wih, Whh = g["Wih"], g["Whh"]
        bih, bhh = g["bih"], g["bhh"]
        outs = []
        for W in (Wih, Whh):
            for k in range(3):
                outs.append(W[k * FP:(k + 1) * FP].T.astype(BF))
        for b in (bih, bhh):
            for k in range(3):
                outs.append(b[k * FP:(k + 1) * FP].reshape(1, FP).astype(f32))
        return outs

    def align_parts(a):
        W = a["W"].astype(f32)
        return [W[:, :FP].T.astype(BF), W[:, FP:].T.astype(BF),
                a["b"].reshape(1, 1).astype(f32)]

    wn = params["neighbor_fc"]["W"].astype(f32)
    weights = (
        [lin_w(params["atom_fc"]), row(params["atom_fc"]["b"]),
         wn[:, :AD].T.astype(BF), wn[:, AD:].T.astype(BF),
         row(params["neighbor_fc"]["b"])]
        + align_parts(params["align"][0]) + align_parts(params["align"][1])
        + [lin_w(params["attend"][0]), row(params["attend"][0]["b"]),
           lin_w(params["attend"][1]), row(params["attend"][1]["b"])]
        + gru_parts(params["gru"][0]) + gru_parts(params["gru"][1])
        + align_parts(params["mol_align"])
        + [lin_w(params["mol_attend"]), row(params["mol_attend"]["b"])]
        + gru_parts(params["mol_gru"])
        + [lin_w(params["output"]), row(params["output"]["b"])]
    )

    R = mb * L
    RN = mb * L * NBR
    data_specs = [
        pl.BlockSpec((mb, L, AD), lambda i: (i, 0, 0)),
        pl.BlockSpec((mb, NB, BD), lambda i: (i, 0, 0)),
        pl.BlockSpec((RN, 1), lambda i: (i, 0)),
        pl.BlockSpec((RN, 1), lambda i: (i, 0)),
        pl.BlockSpec((R, 1), lambda i: (i, 0)),
    ]
    w_specs = [pl.BlockSpec(w.shape, lambda i: tuple(0 for _ in w.shape))
               for w in weights]

    out_shapes = (
        jax.ShapeDtypeStruct((Bz, L, FP), f32),
        jax.ShapeDtypeStruct((Bz, 1), f32),
        jax.ShapeDtypeStruct((Bz, FP), f32),
    )
    out_specs = (
        pl.BlockSpec((mb, L, FP), lambda i: (i, 0, 0)),
        pl.BlockSpec((mb, 1), lambda i: (i, 0)),
        pl.BlockSpec((mb, FP), lambda i: (i, 0)),
    )

    body = functools.partial(_fused_body, mb=mb, L=L, NBR=NBR, NB=NB)
    af, pred, molfeat = pl.pallas_call(
        body,
        grid=(grid,),
        in_specs=data_specs + w_specs,
        out_specs=out_specs,
        out_shape=out_shapes,
        compiler_params=pltpu.CompilerParams(
            dimension_semantics=("arbitrary",),
        ),
    )(atom_list.astype(f32), bond_list.astype(f32), aidxt, bidxt, mask, *weights)
    return (af, pred, molfeat)
